# baseline (speedup 1.0000x reference)
import functools

import jax
import jax.numpy as jnp
from jax import lax
from jax.experimental import pallas as pl
from jax.experimental.pallas import tpu as pltpu
from jax.experimental.pallas import tpu_sc as plsc

SEQ_LEN = 512
HIDDEN = 768
EPS = 1e-12
NUM_WORKERS = 32
BPW = SEQ_LEN // NUM_WORKERS
ROW_BLOCK = 512


def _gather_body(x_hbm, word_hbm, out_hbm, idx_v, rows_v, sem):
    wid = lax.axis_index("s") * 2 + lax.axis_index("c")
    base = wid * BPW
    pltpu.sync_copy(x_hbm.at[pl.ds(base, BPW)], idx_v)
    pltpu.async_copy(word_hbm.at[idx_v], rows_v, sem).wait()
    pltpu.sync_copy(rows_v, out_hbm.at[pl.ds(base, BPW)])


def _ln_body(we_ref, pos_ref, te_ref, gam_ref, bet_ref, o_ref):
    v = we_ref[...] + pos_ref[...] + te_ref[0:1]
    m = jnp.mean(v, axis=-1, keepdims=True)
    c = v - m
    var = jnp.mean(c * c, axis=-1, keepdims=True)
    o_ref[...] = c * lax.rsqrt(var + EPS) * gam_ref[...][None] + bet_ref[...][None]


@jax.jit
def _run(x, word_emb, token_emb, pos_emb, gamma, beta):
    mesh = plsc.VectorSubcoreMesh(core_axis_name="c", subcore_axis_name="s")
    we = pl.kernel(
        _gather_body,
        out_type=jax.ShapeDtypeStruct((SEQ_LEN, HIDDEN), jnp.float32),
        mesh=mesh,
        compiler_params=pltpu.CompilerParams(needs_layout_passes=False),
        scratch_types=[
            pltpu.VMEM((BPW,), jnp.int32),
            pltpu.VMEM((BPW, HIDDEN), jnp.float32),
            pltpu.SemaphoreType.DMA,
        ],
    )(x, word_emb)

    row_spec = pl.BlockSpec((ROW_BLOCK, HIDDEN), lambda i: (i, 0))
    row0_spec = pl.BlockSpec((2, HIDDEN), lambda i: (0, 0))
    vec_spec = pl.BlockSpec((HIDDEN,), lambda i: (0,))
    return pl.pallas_call(
        _ln_body,
        grid=(SEQ_LEN // ROW_BLOCK,),
        in_specs=[row_spec, row_spec, row0_spec, vec_spec, vec_spec],
        out_specs=row_spec,
        out_shape=jax.ShapeDtypeStruct((SEQ_LEN, HIDDEN), jnp.float32),
    )(we, pos_emb, token_emb, gamma, beta)


def kernel(x, word_emb, token_emb, pos_emb, gamma, beta):
    return _run(x.astype(jnp.int32), word_emb, token_emb, pos_emb, gamma, beta)


# speedup vs baseline: 1.0056x; 1.0056x over previous
import functools

import jax
import jax.numpy as jnp
from jax import lax
from jax.experimental import pallas as pl
from jax.experimental.pallas import tpu as pltpu
from jax.experimental.pallas import tpu_sc as plsc

SEQ_LEN = 512
HIDDEN = 768
EPS = 1e-12
NUM_WORKERS = 32
BPW = SEQ_LEN // NUM_WORKERS
ROW_BLOCK = 256


def _gather_body(x_hbm, word_hbm, out_hbm, idx_v, rows_v, sem):
    wid = lax.axis_index("s") * 2 + lax.axis_index("c")
    base = wid * BPW
    pltpu.sync_copy(x_hbm.at[pl.ds(base, BPW)], idx_v)
    pltpu.async_copy(word_hbm.at[idx_v], rows_v, sem).wait()
    pltpu.sync_copy(rows_v, out_hbm.at[pl.ds(base, BPW)])


def _ln_body(we_ref, pos_ref, te_ref, gam_ref, bet_ref, o_ref):
    v = we_ref[...] + pos_ref[...] + te_ref[0:1]
    m = jnp.mean(v, axis=-1, keepdims=True)
    c = v - m
    var = jnp.mean(c * c, axis=-1, keepdims=True)
    o_ref[...] = c * lax.rsqrt(var + EPS) * gam_ref[...][None] + bet_ref[...][None]


@jax.jit
def _run(x, word_emb, token_emb, pos_emb, gamma, beta):
    mesh = plsc.VectorSubcoreMesh(core_axis_name="c", subcore_axis_name="s")
    we = pl.kernel(
        _gather_body,
        out_type=jax.ShapeDtypeStruct((SEQ_LEN, HIDDEN), jnp.float32),
        mesh=mesh,
        compiler_params=pltpu.CompilerParams(needs_layout_passes=False),
        scratch_types=[
            pltpu.VMEM((BPW,), jnp.int32),
            pltpu.VMEM((BPW, HIDDEN), jnp.float32),
            pltpu.SemaphoreType.DMA,
        ],
    )(x, word_emb)

    row_spec = pl.BlockSpec((ROW_BLOCK, HIDDEN), lambda i: (i, 0))
    row0_spec = pl.BlockSpec((2, HIDDEN), lambda i: (0, 0))
    vec_spec = pl.BlockSpec((HIDDEN,), lambda i: (0,))
    return pl.pallas_call(
        _ln_body,
        grid=(SEQ_LEN // ROW_BLOCK,),
        in_specs=[row_spec, row_spec, row0_spec, vec_spec, vec_spec],
        out_specs=row_spec,
        out_shape=jax.ShapeDtypeStruct((SEQ_LEN, HIDDEN), jnp.float32),
    )(we, pos_emb, token_emb, gamma, beta)


def kernel(x, word_emb, token_emb, pos_emb, gamma, beta):
    return _run(x.astype(jnp.int32), word_emb, token_emb, pos_emb, gamma, beta)
